# baseline (speedup 1.0000x reference)
import functools

import jax
import jax.numpy as jnp
from jax import lax
from jax.experimental import pallas as pl
from jax.experimental.pallas import tpu as pltpu
from jax.experimental.pallas import tpu_sc as plsc

B = 32768
K = 8192
D = 32
BB = 256
KC = 512
COMMITMENT = 0.25


def _argmin_body(x_ref, cb_ref, idx_ref, loss_ref):
    i = pl.program_id(0)
    x = x_ref[...]
    a = jnp.sum(x * x, axis=1, keepdims=True)

    def body(k, carry):
        bv, bi = carry
        cbk = cb_ref[pl.ds(k * KC, KC), :]
        bk = jnp.sum(cbk * cbk, axis=1)
        m = lax.dot_general(x, cbk, (((1,), (1,)), ((), ())),
                            preferred_element_type=jnp.float32)
        d = (a + bk[None, :]) - 2.0 * m
        lv = jnp.min(d, axis=1, keepdims=True)
        li = jnp.argmin(d, axis=1).astype(jnp.int32).reshape(BB, 1) + k * KC
        upd = lv < bv
        return jnp.where(upd, lv, bv), jnp.where(upd, li, bi)

    bv0 = jnp.full((BB, 1), jnp.inf, dtype=jnp.float32)
    bi0 = jnp.zeros((BB, 1), dtype=jnp.int32)
    bv, bi = lax.fori_loop(0, K // KC, body, (bv0, bi0))
    idx_ref[...] = bi[:, 0]

    @pl.when(i == 0)
    def _():
        loss_ref[...] = jnp.zeros((1, 1), dtype=jnp.float32)

    loss_ref[...] += jnp.sum(bv).reshape(1, 1)


_dist_argmin = pl.pallas_call(
    _argmin_body,
    grid=(B // BB,),
    in_specs=[
        pl.BlockSpec((BB, D), lambda i: (i, 0)),
        pl.BlockSpec((K, D), lambda i: (0, 0)),
    ],
    out_specs=[
        pl.BlockSpec((BB,), lambda i: (i,)),
        pl.BlockSpec((1, 1), lambda i: (0, 0)),
    ],
    out_shape=[
        jax.ShapeDtypeStruct((B,), jnp.int32),
        jax.ShapeDtypeStruct((1, 1), jnp.float32),
    ],
)


_NW = 32
_NCORES = 2
_BPW = B // _NW
_CH = 128
_NCH = _BPW // _CH


@functools.cache
def _make_gather():
    mesh = plsc.VectorSubcoreMesh(core_axis_name="c", subcore_axis_name="s")

    @functools.partial(
        pl.kernel,
        mesh=mesh,
        out_type=jax.ShapeDtypeStruct((_NW, _NCH, _CH, D), jnp.float32),
        scratch_types=[
            pltpu.VMEM((_NCH, _CH), jnp.int32),
            pltpu.VMEM((_NCH, _CH, D), jnp.float32),
            pltpu.SemaphoreType.DMA,
        ],
        compiler_params=pltpu.CompilerParams(use_tc_tiling_on_sc=False),
    )
    def _gather_body(cb_hbm, idx_hbm, out_hbm, idx_v, rows_v, sem):
        wid = lax.axis_index("s") * _NCORES + lax.axis_index("c")
        pltpu.sync_copy(idx_hbm.at[wid], idx_v)
        copies = [
            pltpu.async_copy(cb_hbm.at[idx_v.at[j]], rows_v.at[j], sem)
            for j in range(_NCH)
        ]
        for cp in copies:
            cp.wait()
        pltpu.sync_copy(rows_v, out_hbm.at[wid])

    return _gather_body


def kernel(inputs, codebook):
    idx, loss_acc = _dist_argmin(inputs, codebook)
    rows = _make_gather()(codebook, idx.reshape(_NW, _NCH, _CH))
    quantized = rows.reshape(B, D)
    mean_sq = loss_acc[0, 0] / (B * D)
    loss = mean_sq + COMMITMENT * mean_sq
    quantized_st = inputs + (quantized - inputs)
    return quantized_st, loss
